# baseline (speedup 1.0000x reference)
import functools

import jax
import jax.numpy as jnp
from jax import lax
from jax.experimental import pallas as pl
from jax.experimental.pallas import tpu as pltpu
from jax.experimental.pallas import tpu_sc as plsc


def _sc_call(b, d, h, w, row_embed, col_embed):
    mesh = plsc.VectorSubcoreMesh(core_axis_name="c", subcore_axis_name="s")
    n_workers = 32
    y_groups = n_workers // b
    rows_per_w = h // y_groups
    lanes = 16

    @functools.partial(
        pl.kernel,
        out_type=jax.ShapeDtypeStruct((b, h, w, 2 * d), jnp.float32),
        mesh=mesh,
        scratch_types=[
            pltpu.VMEM((w, 2 * d), jnp.float32),
            pltpu.VMEM((w, 2 * d), jnp.float32),
            pltpu.VMEM((rows_per_w, d), jnp.float32),
            pltpu.SemaphoreType.DMA,
        ],
    )
    def sck(row_hbm, col_hbm, out_hbm, face0, face1, rbuf, sem):
        wid = lax.axis_index("s") * 2 + lax.axis_index("c")
        b_i = wid // y_groups
        y0 = (wid % y_groups) * rows_per_w
        faces = [face0, face1]
        pltpu.sync_copy(col_hbm.at[pl.ds(0, w), :], face0.at[:, pl.ds(0, d)])
        pltpu.sync_copy(col_hbm.at[pl.ds(0, w), :], face1.at[:, pl.ds(0, d)])
        pltpu.sync_copy(row_hbm.at[pl.ds(y0, rows_per_w), :], rbuf)
        handles = []
        for yi in range(rows_per_w):
            f = faces[yi % 2]
            if yi >= 2:
                handles[yi - 2].wait()
            for j in range(d // lanes):
                v = rbuf[yi, pl.ds(lanes * j, lanes)]
                for x in range(w):
                    f[x, pl.ds(d + lanes * j, lanes)] = v
            cp = pltpu.make_async_copy(f, out_hbm.at[b_i, y0 + yi], sem)
            cp.start()
            handles.append(cp)
        for cp in handles[-2:]:
            cp.wait()

    return sck(row_embed, col_embed)


def kernel(tensor, row_embed, col_embed):
    b = tensor.shape[0]
    h, w = tensor.shape[-2], tensor.shape[-1]
    d = row_embed.shape[1]

    out = _sc_call(b, d, h, w, row_embed, col_embed)
    return out.transpose(0, 3, 1, 2)


# speedup vs baseline: 6.0030x; 6.0030x over previous
import jax
import jax.numpy as jnp
from jax.experimental import pallas as pl
from jax.experimental.pallas import tpu as pltpu


def _make_pos_kernel(b, d, h, w):

    def _pos_kernel(row_ref, col_ref, out_ref, scratch_ref, sem):
        col = col_ref[0:w, :]
        row = row_ref[0:h, :]
        scratch_ref[:, :, 0:d] = jnp.broadcast_to(col[None, :, :], (h, w, d))
        scratch_ref[:, :, d:2 * d] = jnp.broadcast_to(
            row[:, None, :], (h, w, d))
        copies = [
            pltpu.make_async_copy(scratch_ref, out_ref.at[i], sem.at[i])
            for i in range(b)
        ]
        for c in copies:
            c.start()
        for c in copies:
            c.wait()

    return _pos_kernel


def kernel(tensor, row_embed, col_embed):
    b = tensor.shape[0]
    h, w = tensor.shape[-2], tensor.shape[-1]
    d = row_embed.shape[1]

    out = pl.pallas_call(
        _make_pos_kernel(b, d, h, w),
        in_specs=[
            pl.BlockSpec(row_embed.shape, lambda: (0, 0)),
            pl.BlockSpec(col_embed.shape, lambda: (0, 0)),
        ],
        out_specs=pl.BlockSpec(memory_space=pl.ANY),
        out_shape=jax.ShapeDtypeStruct((b, h, w, 2 * d), jnp.float32),
        scratch_shapes=[
            pltpu.VMEM((h, w, 2 * d), jnp.float32),
            pltpu.SemaphoreType.DMA((b,)),
        ],
    )(row_embed, col_embed)
    return out.transpose(0, 3, 1, 2)
